# baseline (speedup 1.0000x reference)
import functools

import jax
import jax.numpy as jnp
from jax import lax
from jax.experimental import pallas as pl
from jax.experimental.pallas import tpu as pltpu
from jax.experimental.pallas import tpu_sc as plsc

NUM_EMBEDDINGS = 1000000
D = 64
BATCH = 16384
HIST = 20
B = BATCH * HIST

NC = 2
NS = 16
NW = NC * NS

_mesh = plsc.VectorSubcoreMesh(
    core_axis_name="c", subcore_axis_name="s", num_cores=NC, num_subcores=NS
)


NBLK = 3906
NB_T = 123


@functools.partial(
    pl.kernel,
    out_type=jax.ShapeDtypeStruct((NUM_EMBEDDINGS // 2, 128), jnp.float32),
    mesh=_mesh,
    compiler_params=pltpu.CompilerParams(
        use_tc_tiling_on_sc=True, needs_layout_passes=False
    ),
    scratch_types=[
        pltpu.VMEM((2, 64, 256), jnp.float32),
        pltpu.VMEM((2, 128, 128), jnp.float32),
        pltpu.SemaphoreType.DMA((2,)),
        pltpu.SemaphoreType.DMA((2,)),
    ],
)
def _transpose_kernel(wt_hbm, tail_hbm, t128_hbm, wbuf, tbuf, isem, osem):
    wid = lax.axis_index("s") * NC + lax.axis_index("c")

    iota = jax.lax.iota(jnp.int32, 16)
    half = iota >> 1
    par64 = (iota & 1) * 64

    def in_copy(m, bi):
        return pltpu.make_async_copy(
            wt_hbm.at[:, pl.ds(256 * m, 256)], wbuf.at[bi], isem.at[bi]
        )

    def out_copy(m, bi):
        return pltpu.make_async_copy(
            tbuf.at[bi], t128_hbm.at[pl.ds(128 * m, 128)], osem.at[bi]
        )

    def do_transpose(bi, nt):
        @plsc.parallel_loop(0, 64, unroll=2)
        def _feat(cp):
            colv = par64 + cp
            vs = [wbuf[bi, cp, pl.ds(16 * t, 16)] for t in range(nt)]
            for t in range(nt):
                rowv = half + (8 * t)
                plsc.store_scatter(tbuf.at[bi], [rowv, colv], vs[t])

    @pl.when(wid < NBLK)
    def _prime():
        in_copy(wid, 0).start()

    @pl.loop(0, 124, step=2)
    def _blk(k0):
        for bi in range(2):
            k = k0 + bi
            m = wid + 32 * k

            @pl.when(m < NBLK)
            def _body():
                in_copy(m, bi).wait()
                nm = m + 32

                @pl.when(nm < NBLK)
                def _next():
                    in_copy(nm, 1 - bi).start()

                @pl.when(k >= 2)
                def _drain():
                    out_copy(m - 64, bi).wait()

                do_transpose(bi, 16)
                out_copy(m, bi).start()

    @pl.when(wid < 2)
    def _drain_a():
        out_copy(wid + 32 * 121, 1).wait()
        out_copy(wid + 32 * 122, 0).wait()

    @pl.when(wid >= 2)
    def _drain_b():
        out_copy(wid + 32 * 120, 0).wait()
        out_copy(wid + 32 * 121, 1).wait()

    @pl.when(wid == 31)
    def _rem():
        pltpu.sync_copy(tail_hbm, tbuf.at[0, pl.ds(0, 32)])
        pltpu.sync_copy(
            tbuf.at[0, pl.ds(0, 32)], t128_hbm.at[pl.ds(128 * NBLK, 32)]
        )


BPW = B // NW
CW = 128
NCHUNK = BPW // CW
NBUF = 8


@functools.partial(
    pl.kernel,
    out_type=jax.ShapeDtypeStruct((B, D), jnp.float32),
    mesh=_mesh,
    compiler_params=pltpu.CompilerParams(use_tc_tiling_on_sc=False),
    scratch_types=[
        pltpu.VMEM((NCHUNK, CW), jnp.int32),
        pltpu.VMEM((NBUF, CW, D), jnp.float32),
        pltpu.SemaphoreType.DMA((NBUF,)),
        pltpu.SemaphoreType.DMA((NBUF,)),
    ],
)
def _gather_kernel(idx_hbm, table_hbm, out_hbm, idx_v, bufs, gsem, ssem):
    wid = lax.axis_index("s") * NC + lax.axis_index("c")
    row0 = wid * NCHUNK
    out0 = wid * BPW

    pltpu.sync_copy(idx_hbm.at[pl.ds(row0, NCHUNK)], idx_v)

    for b in range(NBUF):
        pltpu.async_copy(table_hbm.at[idx_v.at[b]], bufs.at[b], gsem.at[b])

    @pl.loop(0, NCHUNK, step=NBUF)
    def _round(i):
        for b in range(NBUF):
            c = i + b
            pltpu.make_async_copy(
                table_hbm.at[idx_v.at[c]], bufs.at[b], gsem.at[b]
            ).wait()
            dst = out_hbm.at[pl.ds(out0 + c * CW, CW)]
            pltpu.async_copy(bufs.at[b], dst, ssem.at[b])
            nc = c + NBUF

            @pl.when(nc < NCHUNK)
            def _refill():
                pltpu.make_async_copy(bufs.at[b], dst, ssem.at[b]).wait()
                pltpu.async_copy(
                    table_hbm.at[idx_v.at[nc]], bufs.at[b], gsem.at[b]
                )

    for b in range(NBUF):
        c = NCHUNK - NBUF + b
        pltpu.make_async_copy(
            bufs.at[b], out_hbm.at[pl.ds(out0 + c * CW, CW)], ssem.at[b]
        ).wait()


def kernel(token_ids, weight):
    idx = token_ids.reshape(B // CW, CW)
    tail = weight[2 * 128 * NBLK:].reshape(32, 128)
    t128 = _transpose_kernel(weight.T, tail)
    out = _gather_kernel(idx, t128.reshape(NUM_EMBEDDINGS, D))
    return out.reshape(BATCH, HIST, D)


# speedup vs baseline: 1.6338x; 1.6338x over previous
import functools

import jax
import jax.numpy as jnp
from jax import lax
from jax.experimental import pallas as pl
from jax.experimental.pallas import tpu as pltpu
from jax.experimental.pallas import tpu_sc as plsc

NUM_EMBEDDINGS = 1000000
D = 64
BATCH = 16384
HIST = 20
B = BATCH * HIST

NC = 2
NS = 16
NW = NC * NS

_mesh = plsc.VectorSubcoreMesh(
    core_axis_name="c", subcore_axis_name="s", num_cores=NC, num_subcores=NS
)


NBLK = 3906
NB_T = 123


@functools.partial(
    pl.kernel,
    out_type=jax.ShapeDtypeStruct((NUM_EMBEDDINGS // 2, 128), jnp.float32),
    mesh=_mesh,
    compiler_params=pltpu.CompilerParams(
        use_tc_tiling_on_sc=True, needs_layout_passes=False
    ),
    scratch_types=[
        pltpu.VMEM((2, 64, 256), jnp.float32),
        pltpu.VMEM((2, 128, 128), jnp.float32),
        pltpu.SemaphoreType.DMA((2,)),
        pltpu.SemaphoreType.DMA((2,)),
    ],
)
def _transpose_kernel(wt_hbm, tail_hbm, t128_hbm, wbuf, tbuf, isem, osem):
    wid = lax.axis_index("s") * NC + lax.axis_index("c")

    iota = jax.lax.iota(jnp.int32, 16)
    half = iota >> 1
    par64 = (iota & 1) * 64

    def in_copy(m, bi):
        return pltpu.make_async_copy(
            wt_hbm.at[:, pl.ds(256 * m, 256)], wbuf.at[bi], isem.at[bi]
        )

    def out_copy(m, bi):
        return pltpu.make_async_copy(
            tbuf.at[bi], t128_hbm.at[pl.ds(128 * m, 128)], osem.at[bi]
        )

    def do_transpose(bi, nt):
        @plsc.parallel_loop(0, 64, unroll=2)
        def _feat(cp):
            colv = par64 + cp
            vs = [wbuf[bi, cp, pl.ds(16 * t, 16)] for t in range(nt)]
            for t in range(nt):
                rowv = half + (8 * t)
                plsc.store_scatter(tbuf.at[bi], [rowv, colv], vs[t])

    @pl.when(wid < NBLK)
    def _prime():
        in_copy(wid, 0).start()

    @pl.loop(0, 124, step=2)
    def _blk(k0):
        for bi in range(2):
            k = k0 + bi
            m = wid + 32 * k

            @pl.when(m < NBLK)
            def _body():
                in_copy(m, bi).wait()
                nm = m + 32

                @pl.when(nm < NBLK)
                def _next():
                    in_copy(nm, 1 - bi).start()

                @pl.when(k >= 2)
                def _drain():
                    out_copy(m - 64, bi).wait()

                do_transpose(bi, 16)
                out_copy(m, bi).start()

    @pl.when(wid < 2)
    def _drain_a():
        out_copy(wid + 32 * 121, 1).wait()
        out_copy(wid + 32 * 122, 0).wait()

    @pl.when(wid >= 2)
    def _drain_b():
        out_copy(wid + 32 * 120, 0).wait()
        out_copy(wid + 32 * 121, 1).wait()

    @pl.when(wid == 31)
    def _rem():
        pltpu.sync_copy(tail_hbm, tbuf.at[0, pl.ds(0, 32)])
        pltpu.sync_copy(
            tbuf.at[0, pl.ds(0, 32)], t128_hbm.at[pl.ds(128 * NBLK, 32)]
        )


BPW = B // NW
CW = 128
NCHUNK = BPW // CW
NBUF = 8


@functools.partial(
    pl.kernel,
    out_type=jax.ShapeDtypeStruct((B, D), jnp.float32),
    mesh=_mesh,
    compiler_params=pltpu.CompilerParams(
        use_tc_tiling_on_sc=False,
        skip_device_barrier=True,
        disable_bounds_checks=True,
        disable_semaphore_checks=True,
    ),
    scratch_types=[
        pltpu.VMEM((NCHUNK, CW), jnp.int32),
        pltpu.VMEM((NBUF, CW, D), jnp.float32),
        pltpu.SemaphoreType.DMA((NBUF,)),
        pltpu.SemaphoreType.DMA((NBUF,)),
    ],
)
def _gather_kernel(idx_hbm, table_hbm, out_hbm, idx_v, bufs, gsem, ssem):
    wid = lax.axis_index("s") * NC + lax.axis_index("c")
    row0 = wid * NCHUNK
    out0 = wid * BPW

    pltpu.sync_copy(idx_hbm.at[pl.ds(row0, NCHUNK)], idx_v)

    for b in range(NBUF):
        pltpu.async_copy(table_hbm.at[idx_v.at[b]], bufs.at[b], gsem.at[b])

    @pl.loop(0, NCHUNK, step=NBUF)
    def _round(i):
        for b in range(NBUF):
            c = i + b
            pltpu.make_async_copy(
                table_hbm.at[idx_v.at[c]], bufs.at[b], gsem.at[b]
            ).wait()
            dst = out_hbm.at[pl.ds(out0 + c * CW, CW)]
            pltpu.async_copy(bufs.at[b], dst, ssem.at[b])
            nc = c + NBUF

            @pl.when(nc < NCHUNK)
            def _refill():
                pltpu.make_async_copy(bufs.at[b], dst, ssem.at[b]).wait()
                pltpu.async_copy(
                    table_hbm.at[idx_v.at[nc]], bufs.at[b], gsem.at[b]
                )

    for b in range(NBUF):
        c = NCHUNK - NBUF + b
        pltpu.make_async_copy(
            bufs.at[b], out_hbm.at[pl.ds(out0 + c * CW, CW)], ssem.at[b]
        ).wait()


def kernel(token_ids, weight):
    idx = token_ids.reshape(B // CW, CW)
    out = _gather_kernel(idx, weight)
    return out.reshape(BATCH, HIST, D)
